# Initial kernel scaffold; baseline (speedup 1.0000x reference)
import jax
import jax.numpy as jnp
from jax.experimental import pallas as pl


def kernel(x, edge_index, subset_indices, W0, b0, Wm, bm, W_last, b_last):
    raise NotImplementedError("write your pallas kernel here")


# speedup vs baseline: 6.0729x; 6.0729x over previous
import functools

import jax
import jax.numpy as jnp
from jax import lax
from jax.experimental import pallas as pl
from jax.experimental.pallas import tpu as pltpu
from jax.experimental.pallas import tpu_sc as plsc

N = 10000
E = 320000
D = 128
H = 50
NUM_LAYERS = 20
S = 1000

HP = 64
NC, NS, NWK = 2, 16, 32
KE = 128
NB = 84
NBUF = 6
TILE_E = NB * KE
E2P = NWK * TILE_E
E2 = E + N
PADROWS = 128
NPAD = 10240
ZROWS = NPAD // NS
OROWS = 632
NOUT = NS * OROWS
SP = 1024


def _make_agg(width, do_gather):
  mesh = plsc.VectorSubcoreMesh(core_axis_name="c", subcore_axis_name="s")
  nsub = width // 16

  def body(g_hbm, src_hbm, dst_hbm, out_hbm, srcb, dstb, rows, gsems, ssems, acc_ref):
    zeros16 = jnp.zeros((16,), jnp.float32)
    ones16 = jnp.ones((16,), jnp.float32)
    c = lax.axis_index("c")
    s = lax.axis_index("s")
    wid = s * NC + c

    def zbody(i, _):
      for j in range(nsub):
        rows[0][i, pl.ds(j * 16, 16)] = zeros16
      return _
    lax.fori_loop(0, KE, zbody, None)
    for j in range(ZROWS // KE):
      pltpu.sync_copy(rows[0], acc_ref.at[pl.ds(s * ZROWS + j * KE, KE)])
    pltpu.sync_copy(src_hbm.at[wid], srcb)
    pltpu.sync_copy(dst_hbm.at[wid], dstb)
    if not do_gather:
      def obody(i, _):
        for j in range(nsub):
          rows[0][i, pl.ds(j * 16, 16)] = ones16
        return _
      lax.fori_loop(0, KE, obody, None)
    plsc.subcore_barrier()

    def fire_g(i, b):
      pltpu.async_copy(g_hbm.at[srcb.at[i]], rows[b], gsems[b])

    def wait_g(i, b):
      pltpu.make_async_copy(g_hbm.at[srcb.at[i]], rows[b], gsems[b]).wait()

    def fire_s(i, b):
      pltpu.async_copy(rows[b], acc_ref.at[dstb.at[i]], ssems[b], add=True)

    def wait_s(i, b):
      pltpu.make_async_copy(rows[b], acc_ref.at[dstb.at[i]], ssems[b]).wait()

    if do_gather:
      for i in range(4):
        fire_g(i, i % NBUF)
      for i in range(2):
        wait_g(i, i % NBUF)
        fire_s(i, i % NBUF)
        fire_g(i + 4, (i + 4) % NBUF)

      def hexad(m, _):
        for p in range(NBUF):
          i = 2 + NBUF * m + p
          b = (2 + p) % NBUF
          wait_g(i, b)
          fire_s(i, b)
          wait_s(i - 2, (b - 2) % NBUF)
          fire_g(i + 4, (b + 4) % NBUF)
        return _
      lax.fori_loop(0, (NB - 6) // NBUF, hexad, None)
      for i in range(NB - 4, NB):
        b = i % NBUF
        wait_g(i, b)
        fire_s(i, b)
      for i in range(NB - 6, NB):
        wait_s(i, i % NBUF)
    else:
      def sfire(i, _):
        pltpu.async_copy(rows[0], acc_ref.at[dstb.at[i]], ssems[0], add=True)
        return _
      lax.fori_loop(0, NB, sfire, None)
      def sdrain(i, _):
        pltpu.make_async_copy(rows[0], acc_ref.at[dstb.at[i]], ssems[0]).wait()
        return _
      lax.fori_loop(0, NB, sdrain, None)

    plsc.subcore_barrier()
    pltpu.sync_copy(acc_ref.at[pl.ds(s * OROWS, OROWS)],
                    out_hbm.at[c, pl.ds(s * OROWS, OROWS)])

  scratch = [
      pltpu.VMEM((NB, KE), jnp.int32),
      pltpu.VMEM((NB, KE), jnp.int32),
      [pltpu.VMEM((KE, width), jnp.float32) for _ in range(NBUF)],
      [pltpu.SemaphoreType.DMA for _ in range(NBUF)],
      [pltpu.SemaphoreType.DMA for _ in range(NBUF)],
      pltpu.VMEM_SHARED((NPAD, width), jnp.float32),
  ]
  out_t = jax.ShapeDtypeStruct((NC, NOUT, width), jnp.float32)
  cp = pltpu.CompilerParams(use_tc_tiling_on_sc=False)

  if do_gather:
    return pl.kernel(body, out_type=out_t, mesh=mesh, scratch_types=scratch,
                     compiler_params=cp, name=f"sc_agg{width}")
  def body_nog(src_hbm, dst_hbm, out_hbm, *rest):
    return body(None, src_hbm, dst_hbm, out_hbm, *rest)
  return pl.kernel(body_nog, out_type=out_t, mesh=mesh, scratch_types=scratch,
                   compiler_params=cp, name=f"sc_deg{width}")


_agg64 = _make_agg(HP, True)
_agg16 = _make_agg(16, True)
_deg16 = _make_agg(16, False)


def _k_first(x_ref, w_ref, dg0_ref, dg1_ref, g_ref, d_ref):
  deg = dg0_ref[0:N, 0:1] + dg1_ref[0:N, 0:1]
  d = jnp.where(deg > 0, lax.rsqrt(deg), 0.0)
  g_ref[...] = d * jnp.dot(x_ref[...], w_ref[...],
                           preferred_element_type=jnp.float32)
  d_ref[...] = d


def _k_mid(p0_ref, p1_ref, d_ref, b_ref, w_ref, g_ref):
  d = d_ref[...]
  h = jnp.maximum(d * (p0_ref[0:N] + p1_ref[0:N]) + b_ref[...].reshape(1, HP), 0.0)
  g_ref[...] = d * jnp.dot(h, w_ref[...], preferred_element_type=jnp.float32)


def _k_tail(p0_ref, p1_ref, d_ref, b_ref, wl_ref, q_ref):
  d = d_ref[...]
  h = jnp.maximum(d * (p0_ref[0:N] + p1_ref[0:N]) + b_ref[...].reshape(1, HP), 0.0)
  wcol = jnp.sum(wl_ref[...], axis=1, keepdims=True)
  q = d * jnp.dot(h, wcol, preferred_element_type=jnp.float32)
  col = lax.broadcasted_iota(jnp.int32, (1, 16), 1)
  q_ref[...] = jnp.where(col == 0, q, 0.0)


def _k_scale(z0_ref, z1_ref, d_ref, bl_ref, r_ref):
  sumb = jnp.sum(bl_ref[...])
  r_ref[...] = d_ref[...] * (z0_ref[0:N] + z1_ref[0:N]) + sumb


def _make_final():
  mesh = plsc.VectorSubcoreMesh(core_axis_name="c", subcore_axis_name="s")

  def body(r_hbm, sub_hbm, outv_hbm, outi_hbm, subb, rb, ov, oi):
    c = lax.axis_index("c")
    s = lax.axis_index("s")

    @pl.when(jnp.logical_and(c == 0, s == 0))
    def _():
      pltpu.sync_copy(sub_hbm, subb)
      for j in range(SP // KE):
        pltpu.sync_copy(r_hbm.at[subb.at[j]], rb.at[pl.ds(j * KE, KE)])
      iot = lax.iota(jnp.int32, 16)
      zi = jnp.zeros((16,), jnp.int32)
      ninf = jnp.full((16,), -jnp.inf, jnp.float32)

      def mbody(i, carry):
        best, bpos = carry
        idxv = i * 16 + iot
        vals = plsc.load_gather(rb, [idxv, zi])
        v = jnp.where(idxv < S, vals, -jnp.inf)
        upd = v > best
        return jnp.where(upd, v, best), jnp.where(upd, idxv, bpos)

      best, bpos = lax.fori_loop(0, SP // 16, mbody, (ninf, zi))
      gmax = jnp.max(best)
      posm = jnp.where(best == gmax, bpos, jnp.int32(1 << 30))
      pmin = jnp.min(posm)
      ov[...] = jnp.broadcast_to(gmax, (16,))
      oi[...] = jnp.broadcast_to(pmin, (16,))
      pltpu.sync_copy(ov, outv_hbm)
      pltpu.sync_copy(oi, outi_hbm)

  scratch = [
      pltpu.VMEM((SP // KE, KE), jnp.int32),
      pltpu.VMEM((SP, 16), jnp.float32),
      pltpu.VMEM((16,), jnp.float32),
      pltpu.VMEM((16,), jnp.int32),
  ]
  out_t = (jax.ShapeDtypeStruct((16,), jnp.float32),
           jax.ShapeDtypeStruct((16,), jnp.int32))
  return pl.kernel(body, out_type=out_t, mesh=mesh, scratch_types=scratch,
                   compiler_params=pltpu.CompilerParams(use_tc_tiling_on_sc=False,
                                                        needs_layout_passes=False),
                   name="sc_final")


_final = _make_final()

def kernel(x, edge_index, subset_indices, W0, b0, Wm, bm, W_last, b_last):
  f32 = jnp.float32
  loop = jnp.arange(N, dtype=jnp.int32)
  npad = E2P - E2
  pad_dst = N + (jnp.arange(npad, dtype=jnp.int32) % PADROWS)
  src = jnp.concatenate([edge_index[0].astype(jnp.int32), loop,
                         jnp.zeros((npad,), jnp.int32)]).reshape(NWK, NB, KE)
  dst = jnp.concatenate([edge_index[1].astype(jnp.int32), loop,
                         pad_dst]).reshape(NWK, NB, KE)
  sub = jnp.concatenate([subset_indices.astype(jnp.int32),
                         jnp.zeros((SP - S,), jnp.int32)]).reshape(SP // KE, KE)
  W0p = jnp.zeros((D, HP), f32).at[:, :H].set(W0)
  b0p = jnp.zeros((HP,), f32).at[:H].set(b0)
  Wmp = jnp.zeros((NUM_LAYERS - 2, HP, HP), f32).at[:, :H, :H].set(Wm)
  bmp = jnp.zeros((NUM_LAYERS - 2, HP), f32).at[:, :H].set(bm)
  Wlp = jnp.zeros((HP, D), f32).at[:H, :].set(W_last)

  degp = _deg16(src, dst)

  g, d = pl.pallas_call(
      _k_first,
      out_shape=(jax.ShapeDtypeStruct((N, HP), f32),
                 jax.ShapeDtypeStruct((N, 1), f32)),
  )(x, W0p, degp[0], degp[1])

  bias = b0p
  for i in range(NUM_LAYERS - 2):
    p = _agg64(g, src, dst)
    g = pl.pallas_call(
        _k_mid, out_shape=jax.ShapeDtypeStruct((N, HP), f32),
    )(p[0], p[1], d, bias, Wmp[i])
    bias = bmp[i]

  p = _agg64(g, src, dst)
  q = pl.pallas_call(
      _k_tail, out_shape=jax.ShapeDtypeStruct((N, 16), f32),
  )(p[0], p[1], d, bias, Wlp)

  z = _agg16(q, src, dst)
  r = pl.pallas_call(
      _k_scale, out_shape=jax.ShapeDtypeStruct((N, 16), f32),
  )(z[0], z[1], d, b_last)

  outv, outi = _final(r, sub)
  return outv[0], outi[0]


# speedup vs baseline: 6.1767x; 1.0171x over previous
import functools

import jax
import jax.numpy as jnp
from jax import lax
from jax.experimental import pallas as pl
from jax.experimental.pallas import tpu as pltpu
from jax.experimental.pallas import tpu_sc as plsc

N = 10000
E = 320000
D = 128
H = 50
NUM_LAYERS = 20
S = 1000

HP = 64
NC, NS, NWK = 2, 16, 32
KE = 128
NB = 84
TILE_E = NB * KE
E2P = NWK * TILE_E
E2 = E + N
PADROWS = 128
NPAD = 10240
ZROWS = NPAD // NS
OROWS = 632
NOUT = NS * OROWS
SP = 1024


def _make_agg(width, do_gather, spmem_stage):
  mesh = plsc.VectorSubcoreMesh(core_axis_name="c", subcore_axis_name="s")
  nsub = width // 16
  GROWS = N // NS

  def body(g_hbm, src_hbm, dst_hbm, out_hbm, srcb, dstb, r0, r1, s0, s1, acc, gs):
    zeros16 = jnp.zeros((16,), jnp.float32)
    ones16 = jnp.ones((16,), jnp.float32)
    c = lax.axis_index("c")
    s = lax.axis_index("s")
    wid = s * NC + c

    def zbody(i, _):
      for j in range(nsub):
        r0[i, pl.ds(j * 16, 16)] = zeros16
      return _
    lax.fori_loop(0, KE, zbody, None)
    for j in range(ZROWS // KE):
      pltpu.sync_copy(r0, acc.at[pl.ds(s * ZROWS + j * KE, KE)])
    pltpu.sync_copy(src_hbm.at[wid], srcb)
    pltpu.sync_copy(dst_hbm.at[wid], dstb)
    if spmem_stage:
      pltpu.sync_copy(g_hbm.at[pl.ds(s * GROWS, GROWS)], gs.at[pl.ds(s * GROWS, GROWS)])
    if not do_gather:
      def obody(i, _):
        for j in range(nsub):
          r0[i, pl.ds(j * 16, 16)] = ones16
        return _
      lax.fori_loop(0, KE, obody, None)
    plsc.subcore_barrier()

    gsrc = gs if spmem_stage else g_hbm

    if do_gather:
      pltpu.async_copy(gsrc.at[srcb.at[0]], r0, s0)

      def step(i, cur, csem, prev, psem):
        pltpu.async_copy(gsrc.at[srcb.at[i]], cur, csem)
        pltpu.make_async_copy(gsrc.at[srcb.at[i - 1]], prev, psem).wait()
        pltpu.sync_copy(prev, acc.at[dstb.at[i - 1]], add=True)

      def pair(k, _):
        step(2 * k + 1, r1, s1, r0, s0)
        step(2 * k + 2, r0, s0, r1, s1)
        return _
      lax.fori_loop(0, (NB - 2) // 2, pair, None)
      step(NB - 1, r1, s1, r0, s0)
      pltpu.make_async_copy(gsrc.at[srcb.at[NB - 1]], r1, s1).wait()
      pltpu.sync_copy(r1, acc.at[dstb.at[NB - 1]], add=True)
    else:
      pltpu.async_copy(r0, acc.at[dstb.at[0]], s0, add=True)

      def dstep(i, _):
        pltpu.async_copy(r0, acc.at[dstb.at[i]], s0, add=True)
        pltpu.make_async_copy(r0, acc.at[dstb.at[i - 1]], s0).wait()
        return _
      lax.fori_loop(1, NB, dstep, None)
      pltpu.make_async_copy(r0, acc.at[dstb.at[NB - 1]], s0).wait()

    plsc.subcore_barrier()
    pltpu.sync_copy(acc.at[pl.ds(s * OROWS, OROWS)],
                    out_hbm.at[c, pl.ds(s * OROWS, OROWS)])

  scratch = [
      pltpu.VMEM((NB, KE), jnp.int32),
      pltpu.VMEM((NB, KE), jnp.int32),
      pltpu.VMEM((KE, width), jnp.float32),
      pltpu.VMEM((KE, width), jnp.float32),
      pltpu.SemaphoreType.DMA,
      pltpu.SemaphoreType.DMA,
      pltpu.VMEM_SHARED((NPAD, width), jnp.float32),
      pltpu.VMEM_SHARED((N if spmem_stage else 8, width), jnp.float32),
  ]
  out_t = jax.ShapeDtypeStruct((NC, NOUT, width), jnp.float32)
  cp = pltpu.CompilerParams(use_tc_tiling_on_sc=False)

  if do_gather:
    return pl.kernel(body, out_type=out_t, mesh=mesh, scratch_types=scratch,
                     compiler_params=cp, name=f"sc_agg{width}")
  def body_nog(src_hbm, dst_hbm, out_hbm, *rest):
    return body(None, src_hbm, dst_hbm, out_hbm, *rest)
  return pl.kernel(body_nog, out_type=out_t, mesh=mesh, scratch_types=scratch,
                   compiler_params=cp, name=f"sc_deg{width}")


_agg64 = _make_agg(HP, True, False)
_agg16 = _make_agg(16, True, False)
_deg16 = _make_agg(16, False, False)


def _k_first(x_ref, w_ref, dg0_ref, dg1_ref, g_ref, d_ref):
  deg = dg0_ref[0:N, 0:1] + dg1_ref[0:N, 0:1]
  d = jnp.where(deg > 0, lax.rsqrt(deg), 0.0)
  g_ref[...] = d * jnp.dot(x_ref[...], w_ref[...],
                           preferred_element_type=jnp.float32)
  d_ref[...] = d


def _k_mid(p0_ref, p1_ref, d_ref, b_ref, w_ref, g_ref):
  d = d_ref[...]
  h = jnp.maximum(d * (p0_ref[0:N] + p1_ref[0:N]) + b_ref[...].reshape(1, HP), 0.0)
  g_ref[...] = d * jnp.dot(h, w_ref[...], preferred_element_type=jnp.float32)


def _k_tail(p0_ref, p1_ref, d_ref, b_ref, wl_ref, q_ref):
  d = d_ref[...]
  h = jnp.maximum(d * (p0_ref[0:N] + p1_ref[0:N]) + b_ref[...].reshape(1, HP), 0.0)
  wcol = jnp.sum(wl_ref[...], axis=1, keepdims=True)
  q = d * jnp.dot(h, wcol, preferred_element_type=jnp.float32)
  col = lax.broadcasted_iota(jnp.int32, (1, 16), 1)
  q_ref[...] = jnp.where(col == 0, q, 0.0)


def _k_scale(z0_ref, z1_ref, d_ref, bl_ref, r_ref):
  sumb = jnp.sum(bl_ref[...])
  r_ref[...] = d_ref[...] * (z0_ref[0:N] + z1_ref[0:N]) + sumb


def _make_final():
  mesh = plsc.VectorSubcoreMesh(core_axis_name="c", subcore_axis_name="s")

  def body(r_hbm, sub_hbm, outv_hbm, outi_hbm, subb, rb, ov, oi):
    c = lax.axis_index("c")
    s = lax.axis_index("s")

    @pl.when(jnp.logical_and(c == 0, s == 0))
    def _():
      pltpu.sync_copy(sub_hbm, subb)
      for j in range(SP // KE):
        pltpu.sync_copy(r_hbm.at[subb.at[j]], rb.at[pl.ds(j * KE, KE)])
      iot = lax.iota(jnp.int32, 16)
      zi = jnp.zeros((16,), jnp.int32)
      ninf = jnp.full((16,), -jnp.inf, jnp.float32)

      def mbody(i, carry):
        best, bpos = carry
        idxv = i * 16 + iot
        vals = plsc.load_gather(rb, [idxv, zi])
        v = jnp.where(idxv < S, vals, -jnp.inf)
        upd = v > best
        return jnp.where(upd, v, best), jnp.where(upd, idxv, bpos)

      best, bpos = lax.fori_loop(0, SP // 16, mbody, (ninf, zi))
      gmax = jnp.max(best)
      posm = jnp.where(best == gmax, bpos, jnp.int32(1 << 30))
      pmin = jnp.min(posm)
      ov[...] = jnp.broadcast_to(gmax, (16,))
      oi[...] = jnp.broadcast_to(pmin, (16,))
      pltpu.sync_copy(ov, outv_hbm)
      pltpu.sync_copy(oi, outi_hbm)

  scratch = [
      pltpu.VMEM((SP // KE, KE), jnp.int32),
      pltpu.VMEM((SP, 16), jnp.float32),
      pltpu.VMEM((16,), jnp.float32),
      pltpu.VMEM((16,), jnp.int32),
  ]
  out_t = (jax.ShapeDtypeStruct((16,), jnp.float32),
           jax.ShapeDtypeStruct((16,), jnp.int32))
  return pl.kernel(body, out_type=out_t, mesh=mesh, scratch_types=scratch,
                   compiler_params=pltpu.CompilerParams(use_tc_tiling_on_sc=False,
                                                        needs_layout_passes=False),
                   name="sc_final")


_final = _make_final()

def kernel(x, edge_index, subset_indices, W0, b0, Wm, bm, W_last, b_last):
  f32 = jnp.float32
  loop = jnp.arange(N, dtype=jnp.int32)
  npad = E2P - E2
  pad_dst = N + (jnp.arange(npad, dtype=jnp.int32) % PADROWS)
  src = jnp.concatenate([edge_index[0].astype(jnp.int32), loop,
                         jnp.zeros((npad,), jnp.int32)]).reshape(NWK, NB, KE)
  dst = jnp.concatenate([edge_index[1].astype(jnp.int32), loop,
                         pad_dst]).reshape(NWK, NB, KE)
  sub = jnp.concatenate([subset_indices.astype(jnp.int32),
                         jnp.zeros((SP - S,), jnp.int32)]).reshape(SP // KE, KE)
  W0p = jnp.zeros((D, HP), f32).at[:, :H].set(W0)
  b0p = jnp.zeros((HP,), f32).at[:H].set(b0)
  Wmp = jnp.zeros((NUM_LAYERS - 2, HP, HP), f32).at[:, :H, :H].set(Wm)
  bmp = jnp.zeros((NUM_LAYERS - 2, HP), f32).at[:, :H].set(bm)
  Wlp = jnp.zeros((HP, D), f32).at[:H, :].set(W_last)

  degp = _deg16(src, dst)

  g, d = pl.pallas_call(
      _k_first,
      out_shape=(jax.ShapeDtypeStruct((N, HP), f32),
                 jax.ShapeDtypeStruct((N, 1), f32)),
  )(x, W0p, degp[0], degp[1])

  bias = b0p
  for i in range(NUM_LAYERS - 2):
    p = _agg64(g, src, dst)
    g = pl.pallas_call(
        _k_mid, out_shape=jax.ShapeDtypeStruct((N, HP), f32),
    )(p[0], p[1], d, bias, Wmp[i])
    bias = bmp[i]

  p = _agg64(g, src, dst)
  q = pl.pallas_call(
      _k_tail, out_shape=jax.ShapeDtypeStruct((N, 16), f32),
  )(p[0], p[1], d, bias, Wlp)

  z = _agg16(q, src, dst)
  r = pl.pallas_call(
      _k_scale, out_shape=jax.ShapeDtypeStruct((N, 16), f32),
  )(z[0], z[1], d, b_last)

  outv, outi = _final(r, sub)
  return outv[0], outi[0]


# speedup vs baseline: 11.4595x; 1.8553x over previous
import functools

import jax
import jax.numpy as jnp
from jax import lax
from jax.experimental import pallas as pl
from jax.experimental.pallas import tpu as pltpu
from jax.experimental.pallas import tpu_sc as plsc

N = 10000
E = 320000
D = 128
H = 50
NUM_LAYERS = 20
S = 1000

HP = 64
NC, NS, NWK = 2, 16, 32
KE = 128
NB = 82
TILE_E = NB * KE
E2P = NWK * TILE_E
E2 = E + N
PADROWS = 128
NPAD = 10240
ZROWS = NPAD // NS
OROWS = 632
NOUT = NS * OROWS
SP = 1024


def _make_agg(width, do_gather):
  mesh = plsc.VectorSubcoreMesh(core_axis_name="c", subcore_axis_name="s")
  nsub = width // 16

  def body(g_hbm, src_hbm, dst_hbm, out_hbm, srcb, dstb, r0, r1, zb, acc, s0, s1):
    zeros16 = jnp.zeros((16,), jnp.float32)
    ones16 = jnp.ones((16,), jnp.float32)
    c = lax.axis_index("c")
    s = lax.axis_index("s")
    wid = s * NC + c

    def zbody(i, _):
      for j in range(nsub):
        zb[i, pl.ds(j * 16, 16)] = zeros16
      return _
    lax.fori_loop(0, ZROWS, zbody, None)
    pltpu.sync_copy(zb, acc.at[pl.ds(s * ZROWS, ZROWS)])
    pltpu.sync_copy(src_hbm.at[wid], srcb)
    pltpu.sync_copy(dst_hbm.at[wid], dstb)
    if not do_gather:
      def obody(i, _):
        for j in range(nsub):
          r0[i, pl.ds(j * 16, 16)] = ones16
        return _
      lax.fori_loop(0, KE, obody, None)
    plsc.subcore_barrier()

    if do_gather:
      pltpu.async_copy(g_hbm.at[srcb.at[0]], r0, s0)

      def step(i, cur, csem, prev, psem):
        pltpu.async_copy(g_hbm.at[srcb.at[i]], cur, csem)
        pltpu.make_async_copy(g_hbm.at[srcb.at[i - 1]], prev, psem).wait()
        pltpu.sync_copy(prev, acc.at[dstb.at[i - 1]], add=True)

      def pair(k, _):
        step(2 * k + 1, r1, s1, r0, s0)
        step(2 * k + 2, r0, s0, r1, s1)
        return _
      lax.fori_loop(0, (NB - 2) // 2, pair, None)
      step(NB - 1, r1, s1, r0, s0)
      pltpu.make_async_copy(g_hbm.at[srcb.at[NB - 1]], r1, s1).wait()
      pltpu.sync_copy(r1, acc.at[dstb.at[NB - 1]], add=True)
    else:
      def sbody(i, _):
        pltpu.sync_copy(r0, acc.at[dstb.at[i]], add=True)
        return _
      lax.fori_loop(0, NB, sbody, None)

    plsc.subcore_barrier()
    pltpu.sync_copy(acc.at[pl.ds(s * OROWS, OROWS)],
                    out_hbm.at[c, pl.ds(s * OROWS, OROWS)])

  scratch = [
      pltpu.VMEM((NB, KE), jnp.int32),
      pltpu.VMEM((NB, KE), jnp.int32),
      pltpu.VMEM((KE, width), jnp.float32),
      pltpu.VMEM((KE, width), jnp.float32),
      pltpu.VMEM((ZROWS, width), jnp.float32),
      pltpu.VMEM_SHARED((NPAD, width), jnp.float32),
      pltpu.SemaphoreType.DMA,
      pltpu.SemaphoreType.DMA,
  ]
  out_t = jax.ShapeDtypeStruct((NC, NOUT, width), jnp.float32)
  cp = pltpu.CompilerParams(use_tc_tiling_on_sc=False)
  if do_gather:
    return pl.kernel(body, out_type=out_t, mesh=mesh, scratch_types=scratch,
                     compiler_params=cp, name=f"sc_agg{width}")
  def body_nog(src_hbm, dst_hbm, out_hbm, *rest):
    return body(None, src_hbm, dst_hbm, out_hbm, *rest)
  return pl.kernel(body_nog, out_type=out_t, mesh=mesh, scratch_types=scratch,
                   compiler_params=cp, name=f"sc_deg{width}")


_agg64 = _make_agg(HP, True)
_agg16 = _make_agg(16, True)
_deg16 = _make_agg(16, False)


def _k_first(x_ref, w_ref, dg0_ref, dg1_ref, g_ref, d_ref):
  deg = dg0_ref[0:N, 0:1] + dg1_ref[0:N, 0:1]
  d = jnp.where(deg > 0, lax.rsqrt(deg), 0.0)
  g_ref[...] = d * jnp.dot(x_ref[...], w_ref[...],
                           preferred_element_type=jnp.float32)
  d_ref[...] = d


def _k_mid(p0_ref, p1_ref, d_ref, b_ref, w_ref, g_ref):
  d = d_ref[...]
  h = jnp.maximum(d * (p0_ref[0:N] + p1_ref[0:N]) + b_ref[...].reshape(1, HP), 0.0)
  g_ref[...] = d * jnp.dot(h, w_ref[...], preferred_element_type=jnp.float32)


def _k_tail(p0_ref, p1_ref, d_ref, b_ref, wl_ref, q_ref):
  d = d_ref[...]
  h = jnp.maximum(d * (p0_ref[0:N] + p1_ref[0:N]) + b_ref[...].reshape(1, HP), 0.0)
  wcol = jnp.sum(wl_ref[...], axis=1, keepdims=True)
  q = d * jnp.dot(h, wcol, preferred_element_type=jnp.float32)
  col = lax.broadcasted_iota(jnp.int32, (1, 16), 1)
  q_ref[...] = jnp.where(col == 0, q, 0.0)


def _k_scale(z0_ref, z1_ref, d_ref, bl_ref, r_ref):
  sumb = jnp.sum(bl_ref[...])
  r_ref[...] = d_ref[...] * (z0_ref[0:N] + z1_ref[0:N]) + sumb


def _make_final():
  mesh = plsc.VectorSubcoreMesh(core_axis_name="c", subcore_axis_name="s")

  def body(r_hbm, sub_hbm, outv_hbm, outi_hbm, subb, rb, ov, oi):
    c = lax.axis_index("c")
    s = lax.axis_index("s")

    @pl.when(jnp.logical_and(c == 0, s == 0))
    def _():
      pltpu.sync_copy(sub_hbm, subb)
      for j in range(SP // KE):
        pltpu.sync_copy(r_hbm.at[subb.at[j]], rb.at[pl.ds(j * KE, KE)])
      iot = lax.iota(jnp.int32, 16)
      zi = jnp.zeros((16,), jnp.int32)
      ninf = jnp.full((16,), -jnp.inf, jnp.float32)

      def mbody(i, carry):
        best, bpos = carry
        idxv = i * 16 + iot
        vals = plsc.load_gather(rb, [idxv, zi])
        v = jnp.where(idxv < S, vals, -jnp.inf)
        upd = v > best
        return jnp.where(upd, v, best), jnp.where(upd, idxv, bpos)

      best, bpos = lax.fori_loop(0, SP // 16, mbody, (ninf, zi))
      gmax = jnp.max(best)
      posm = jnp.where(best == gmax, bpos, jnp.int32(1 << 30))
      pmin = jnp.min(posm)
      ov[...] = jnp.broadcast_to(gmax, (16,))
      oi[...] = jnp.broadcast_to(pmin, (16,))
      pltpu.sync_copy(ov, outv_hbm)
      pltpu.sync_copy(oi, outi_hbm)

  scratch = [
      pltpu.VMEM((SP // KE, KE), jnp.int32),
      pltpu.VMEM((SP, 16), jnp.float32),
      pltpu.VMEM((16,), jnp.float32),
      pltpu.VMEM((16,), jnp.int32),
  ]
  out_t = (jax.ShapeDtypeStruct((16,), jnp.float32),
           jax.ShapeDtypeStruct((16,), jnp.int32))
  return pl.kernel(body, out_type=out_t, mesh=mesh, scratch_types=scratch,
                   compiler_params=pltpu.CompilerParams(use_tc_tiling_on_sc=False,
                                                        needs_layout_passes=False),
                   name="sc_final")


_final = _make_final()

def kernel(x, edge_index, subset_indices, W0, b0, Wm, bm, W_last, b_last):
  f32 = jnp.float32
  loop = jnp.arange(N, dtype=jnp.int32)
  npad = E2P - E2
  pad_dst = N + (jnp.arange(npad, dtype=jnp.int32) % PADROWS)
  src = jnp.concatenate([edge_index[0].astype(jnp.int32), loop,
                         jnp.zeros((npad,), jnp.int32)]).reshape(NWK, NB, KE)
  dst = jnp.concatenate([edge_index[1].astype(jnp.int32), loop,
                         pad_dst]).reshape(NWK, NB, KE)
  sub = jnp.concatenate([subset_indices.astype(jnp.int32),
                         jnp.zeros((SP - S,), jnp.int32)]).reshape(SP // KE, KE)
  W0p = jnp.zeros((D, HP), f32).at[:, :H].set(W0)
  b0p = jnp.zeros((HP,), f32).at[:H].set(b0)
  Wmp = jnp.zeros((NUM_LAYERS - 2, HP, HP), f32).at[:, :H, :H].set(Wm)
  bmp = jnp.zeros((NUM_LAYERS - 2, HP), f32).at[:, :H].set(bm)
  Wlp = jnp.zeros((HP, D), f32).at[:H, :].set(W_last)

  degp = _deg16(src, dst)

  g, d = pl.pallas_call(
      _k_first,
      out_shape=(jax.ShapeDtypeStruct((N, HP), f32),
                 jax.ShapeDtypeStruct((N, 1), f32)),
  )(x, W0p, degp[0], degp[1])

  bias = b0p
  for i in range(NUM_LAYERS - 2):
    p = _agg64(g, src, dst)
    g = pl.pallas_call(
        _k_mid, out_shape=jax.ShapeDtypeStruct((N, HP), f32),
    )(p[0], p[1], d, bias, Wmp[i])
    bias = bmp[i]

  p = _agg64(g, src, dst)
  q = pl.pallas_call(
      _k_tail, out_shape=jax.ShapeDtypeStruct((N, 16), f32),
  )(p[0], p[1], d, bias, Wlp)

  z = _agg16(q, src, dst)
  r = pl.pallas_call(
      _k_scale, out_shape=jax.ShapeDtypeStruct((N, 16), f32),
  )(z[0], z[1], d, b_last)

  outv, outi = _final(r, sub)
  return outv[0], outi[0]
